# baseline (speedup 1.0000x reference)
import functools

import jax
import jax.numpy as jnp
from jax.experimental import pallas as pl

_B = 64
_N = 2048
_S = 16
_RADII = (0.1, 0.2, 0.4)
_NS = (16, 32, 64)


def _stage1_body(xyz_ref, xyzt_ref, o1_ref, o2_ref, o3_ref):
    full = xyz_ref[0]
    fullt = xyzt_ref[0]
    x = full[0:1, :]
    y = full[1:2, :]
    z = full[2:3, :]
    lane = jax.lax.broadcasted_iota(jnp.int32, (1, _N), 1)
    col16 = jax.lax.broadcasted_iota(jnp.int32, (1, _S), 1)
    row16 = jax.lax.broadcasted_iota(jnp.int32, (_S, 1), 0)

    def body(i, c):
        dist, far, nxc, nyc, nzc = c
        sel = lane == far
        cx = jnp.sum(jnp.where(sel, x, 0.0))
        cy = jnp.sum(jnp.where(sel, y, 0.0))
        cz = jnp.sum(jnp.where(sel, z, 0.0))
        nxc = jnp.where(row16 == i, cx, nxc)
        nyc = jnp.where(row16 == i, cy, nyc)
        nzc = jnp.where(row16 == i, cz, nzc)
        dx = x - cx
        dy = y - cy
        dz = z - cz
        d = dx * dx + dy * dy
        d = d + dz * dz
        dist = jnp.minimum(dist, d)
        mx = jnp.max(dist)
        far2 = jnp.min(jnp.where(dist == mx, lane, _N)).astype(jnp.int32)
        return dist, far2, nxc, nyc, nzc

    zc = jnp.zeros((_S, 1), jnp.float32)
    init = (jnp.full((1, _N), 1e10, jnp.float32), jnp.int32(0), zc, zc, zc)
    _, _, nxc, nyc, nzc = jax.lax.fori_loop(0, _S, body, init)
    new_mat_t = jnp.concatenate([nxc, nyc, nzc], axis=1)

    p6t = jnp.concatenate([fullt[:, 3:6], fullt[:, 0:3]], axis=1)

    dx = x - nxc
    dy = y - nyc
    dz = z - nzc
    sq = dx * dx + dy * dy
    sq = sq + dz * dz
    for o_ref, radius, K in ((o1_ref,) + (_RADII[0], _NS[0]),
                             (o2_ref,) + (_RADII[1], _NS[1]),
                             (o3_ref,) + (_RADII[2], _NS[2])):
        mask = sq <= jnp.float32(radius ** 2)
        r = mask.astype(jnp.float32)
        sh = 1
        while sh < _N:
            r = r + jnp.concatenate(
                [jnp.zeros((_S, sh), jnp.float32), r[:, : _N - sh]], axis=1)
            sh *= 2
        count = r[:, _N - 1:_N]
        kv = jax.lax.broadcasted_iota(jnp.int32, (1, K), 1).astype(jnp.float32) + 1.0
        keff = jnp.where(kv <= count, kv, 1.0)
        sel3 = jnp.logical_and(
            r.reshape(_S, 1, _N) == keff.reshape(_S, K, 1),
            mask.reshape(_S, 1, _N))
        selm = sel3.astype(jnp.float32).reshape(_S * K, _N)
        feats = jax.lax.dot_general(
            selm, p6t, (((1,), (0,)), ((), ())),
            precision=jax.lax.Precision.HIGHEST,
            preferred_element_type=jnp.float32)
        expand = (jax.lax.broadcasted_iota(jnp.int32, (_S * K, _S), 0) // K
                  == jax.lax.broadcasted_iota(jnp.int32, (_S * K, _S), 1))
        centers = jax.lax.dot_general(
            expand.astype(jnp.float32), new_mat_t, (((1,), (0,)), ((), ())),
            precision=jax.lax.Precision.HIGHEST,
            preferred_element_type=jnp.float32)
        o_ref[0] = jnp.concatenate(
            [feats[:, 0:3], feats[:, 3:6] - centers], axis=1)


def _mlp_body(K, x_ref, *refs):
    out_ref = refs[-1]
    h = x_ref[...]
    m_cols = h.shape[1]
    for li in range(3):
        w = refs[li * 4][...]
        b = refs[li * 4 + 1][...]
        g = refs[li * 4 + 2][...]
        be = refs[li * 4 + 3][...]
        yv = jax.lax.dot_general(
            w, h, (((1,), (0,)), ((), ())),
            preferred_element_type=jnp.float32) + b
        mu = jnp.mean(yv, axis=1, keepdims=True)
        d = yv - mu
        v = jnp.mean(d * d, axis=1, keepdims=True)
        h = jnp.maximum(d / jnp.sqrt(v + 1e-5) * g + be, 0.0)
    bs = m_cols // K
    p = h[:, 0:bs]
    for k in range(1, K):
        p = jnp.maximum(p, h[:, k * bs:(k + 1) * bs])
    out_ref[...] = p


def _head_body(x_ref, w1_ref, b1_ref, g1_ref, e1_ref,
               w2_ref, b2_ref, g2_ref, e2_ref, out_ref):
    h = x_ref[...]
    y = jax.lax.dot_general(
        w1_ref[...], h, (((1,), (0,)), ((), ())),
        preferred_element_type=jnp.float32) + b1_ref[...]
    mu = jnp.mean(y, axis=1, keepdims=True)
    d = y - mu
    v = jnp.mean(d * d, axis=1, keepdims=True)
    h = jnp.maximum(d / jnp.sqrt(v + 1e-5) * g1_ref[...] + e1_ref[...], 0.0)
    y = jax.lax.dot_general(
        w2_ref[...], h, (((1,), (0,)), ((), ())),
        preferred_element_type=jnp.float32) + b2_ref[...]
    mu = jnp.mean(y, axis=1, keepdims=True)
    d = y - mu
    v = jnp.mean(d * d, axis=1, keepdims=True)
    out_ref[...] = jnp.maximum(
        d / jnp.sqrt(v + 1e-5) * g2_ref[...] + e2_ref[...], 0.0)


def kernel(xyz, params):
    f32 = jnp.float32
    xyzt = jnp.transpose(xyz, (0, 2, 1))
    outs1 = pl.pallas_call(
        _stage1_body,
        grid=(_B,),
        in_specs=[
            pl.BlockSpec((1, 6, _N), lambda b: (b, 0, 0)),
            pl.BlockSpec((1, _N, 6), lambda b: (b, 0, 0)),
        ],
        out_specs=[pl.BlockSpec((1, _S * K, 6), lambda b: (b, 0, 0))
                   for K in _NS],
        out_shape=[jax.ShapeDtypeStruct((_B, _S * K, 6), f32) for K in _NS],
    )(xyz, xyzt)

    pooled_rows = []
    for i, K in enumerate(_NS):
        xin = (outs1[i].reshape(_B, _S, K, 6)
               .transpose(3, 2, 0, 1).reshape(6, K * _B * _S))
        layers = params["convs"][i]
        args = [xin]
        for lyr in layers:
            oc = lyr["w"].shape[0]
            args += [lyr["w"], lyr["b"].reshape(oc, 1),
                     lyr["g"].reshape(oc, 1), lyr["beta"].reshape(oc, 1)]
        c_out = layers[-1]["w"].shape[0]
        pooled = pl.pallas_call(
            functools.partial(_mlp_body, K),
            out_shape=jax.ShapeDtypeStruct((c_out, _B * _S), f32),
        )(*args)
        pooled_rows.append(
            pooled.reshape(c_out, _B, _S).transpose(0, 2, 1)
            .reshape(c_out * _S, _B))
    x1 = jnp.concatenate(pooled_rows, axis=0)

    out = pl.pallas_call(
        _head_body,
        out_shape=jax.ShapeDtypeStruct((256, _B), f32),
    )(x1,
      params["fc1_w"], params["fc1_b"].reshape(64, 1),
      params["bn1_g"].reshape(64, 1), params["bn1_b"].reshape(64, 1),
      params["fc2_w"], params["fc2_b"].reshape(256, 1),
      params["bn2_g"].reshape(256, 1), params["bn2_b"].reshape(256, 1))
    return out.T


# speedup vs baseline: 1.5765x; 1.5765x over previous
import functools

import jax
import jax.numpy as jnp
from jax.experimental import pallas as pl

_B = 64
_N = 2048
_S = 16
_RADII = (0.1, 0.2, 0.4)
_NS = (16, 32, 64)


def _fps_body(coords_ref, outc_ref, outr_ref):
    x = coords_ref[0]
    y = coords_ref[1]
    z = coords_ref[2]
    lane = jax.lax.broadcasted_iota(jnp.int32, (1, _N), 1)
    scol = jax.lax.broadcasted_iota(jnp.int32, (1, _S), 1)

    def body(i, c):
        dist, far, nx, ny, nz = c
        sel = lane == far
        cx = jnp.sum(jnp.where(sel, x, 0.0), axis=1, keepdims=True)
        cy = jnp.sum(jnp.where(sel, y, 0.0), axis=1, keepdims=True)
        cz = jnp.sum(jnp.where(sel, z, 0.0), axis=1, keepdims=True)
        hit = scol == i
        nx = jnp.where(hit, cx, nx)
        ny = jnp.where(hit, cy, ny)
        nz = jnp.where(hit, cz, nz)
        dx = x - cx
        dy = y - cy
        dz = z - cz
        d = dx * dx + dy * dy
        d = d + dz * dz
        dist = jnp.minimum(dist, d)
        mx = jnp.max(dist, axis=1, keepdims=True)
        far2 = jnp.min(jnp.where(dist == mx, lane, _N),
                       axis=1, keepdims=True).astype(jnp.int32)
        return dist, far2, nx, ny, nz

    zi = jnp.zeros((_B, _S), jnp.float32)
    init = (jnp.full((_B, _N), 1e10, jnp.float32),
            jnp.zeros((_B, 1), jnp.int32), zi, zi, zi)
    _, _, nx, ny, nz = jax.lax.fori_loop(0, _S, body, init)
    outc_ref[:, :, 0] = nx
    outc_ref[:, :, 1] = ny
    outc_ref[:, :, 2] = nz
    outr_ref[:, 0, :] = nx
    outr_ref[:, 1, :] = ny
    outr_ref[:, 2, :] = nz


def _stage1_body(xyz_ref, cenc_ref, cenr_ref, o1_ref, o2_ref, o3_ref):
    full = xyz_ref[0]
    cen_c = cenc_ref[0]
    new_mat = cenr_ref[0]
    x = full[0:1, :]
    y = full[1:2, :]
    z = full[2:3, :]
    nxc = cen_c[:, 0:1]
    nyc = cen_c[:, 1:2]
    nzc = cen_c[:, 2:3]

    p6 = jnp.concatenate([full[3:6], full[0:3]], axis=0)

    dx = x - nxc
    dy = y - nyc
    dz = z - nzc
    sq = dx * dx + dy * dy
    sq = sq + dz * dz
    for o_ref, radius, K in ((o1_ref, _RADII[0], _NS[0]),
                             (o2_ref, _RADII[1], _NS[1]),
                             (o3_ref, _RADII[2], _NS[2])):
        mask = sq <= jnp.float32(radius ** 2)
        r = mask.astype(jnp.float32)
        sh = 1
        while sh < _N:
            r = r + jnp.concatenate(
                [jnp.zeros((_S, sh), jnp.float32), r[:, : _N - sh]], axis=1)
            sh *= 2
        count = r[:, _N - 1:_N]
        kv = jax.lax.broadcasted_iota(
            jnp.int32, (1, K), 1).astype(jnp.float32) + 1.0
        keff = jnp.where(kv <= count, kv, 1.0)
        sel3 = jnp.logical_and(
            r.reshape(_S, 1, _N) == keff.reshape(_S, K, 1),
            mask.reshape(_S, 1, _N))
        selm = sel3.astype(jnp.float32).reshape(_S * K, _N)
        feats = jax.lax.dot_general(
            p6, selm, (((1,), (1,)), ((), ())),
            precision=jax.lax.Precision.HIGHEST,
            preferred_element_type=jnp.float32)
        expand = (jax.lax.broadcasted_iota(jnp.int32, (_S, _S * K), 1) // K
                  == jax.lax.broadcasted_iota(jnp.int32, (_S, _S * K), 0))
        centers = jax.lax.dot_general(
            new_mat, expand.astype(jnp.float32), (((1,), (0,)), ((), ())),
            precision=jax.lax.Precision.HIGHEST,
            preferred_element_type=jnp.float32)
        o_ref[0] = jnp.concatenate(
            [feats[0:3], feats[3:6] - centers], axis=0)


def _mlp_body(K, x_ref, *refs):
    out_ref = refs[-1]
    h = x_ref[...]
    m_cols = h.shape[1]
    for li in range(3):
        w = refs[li * 4][...]
        b = refs[li * 4 + 1][...]
        g = refs[li * 4 + 2][...]
        be = refs[li * 4 + 3][...]
        yv = jax.lax.dot_general(
            w, h, (((1,), (0,)), ((), ())),
            preferred_element_type=jnp.float32) + b
        mu = jnp.mean(yv, axis=1, keepdims=True)
        d = yv - mu
        v = jnp.mean(d * d, axis=1, keepdims=True)
        h = jnp.maximum(d / jnp.sqrt(v + 1e-5) * g + be, 0.0)
    bs = m_cols // K
    p = h[:, 0:bs]
    for k in range(1, K):
        p = jnp.maximum(p, h[:, k * bs:(k + 1) * bs])
    out_ref[...] = p


def _head_body(x_ref, w1_ref, b1_ref, g1_ref, e1_ref,
               w2_ref, b2_ref, g2_ref, e2_ref, out_ref):
    h = x_ref[...]
    y = jax.lax.dot_general(
        w1_ref[...], h, (((1,), (0,)), ((), ())),
        preferred_element_type=jnp.float32) + b1_ref[...]
    mu = jnp.mean(y, axis=1, keepdims=True)
    d = y - mu
    v = jnp.mean(d * d, axis=1, keepdims=True)
    h = jnp.maximum(d / jnp.sqrt(v + 1e-5) * g1_ref[...] + e1_ref[...], 0.0)
    y = jax.lax.dot_general(
        w2_ref[...], h, (((1,), (0,)), ((), ())),
        preferred_element_type=jnp.float32) + b2_ref[...]
    mu = jnp.mean(y, axis=1, keepdims=True)
    d = y - mu
    v = jnp.mean(d * d, axis=1, keepdims=True)
    out_ref[...] = jnp.maximum(
        d / jnp.sqrt(v + 1e-5) * g2_ref[...] + e2_ref[...], 0.0)


def kernel(xyz, params):
    f32 = jnp.float32
    coords = jnp.transpose(xyz[:, 0:3, :], (1, 0, 2))
    cen_c, cen_r = pl.pallas_call(
        _fps_body,
        out_shape=[jax.ShapeDtypeStruct((_B, _S, 3), f32),
                   jax.ShapeDtypeStruct((_B, 3, _S), f32)],
    )(coords)

    outs1 = pl.pallas_call(
        _stage1_body,
        grid=(_B,),
        in_specs=[
            pl.BlockSpec((1, 6, _N), lambda b: (b, 0, 0)),
            pl.BlockSpec((1, _S, 3), lambda b: (b, 0, 0)),
            pl.BlockSpec((1, 3, _S), lambda b: (b, 0, 0)),
        ],
        out_specs=[pl.BlockSpec((1, 6, _S * K), lambda b: (b, 0, 0))
                   for K in _NS],
        out_shape=[jax.ShapeDtypeStruct((_B, 6, _S * K), f32) for K in _NS],
    )(xyz, cen_c, cen_r)

    pooled_rows = []
    for i, K in enumerate(_NS):
        xin = (outs1[i].reshape(_B, 6, _S, K)
               .transpose(1, 3, 0, 2).reshape(6, K * _B * _S))
        layers = params["convs"][i]
        args = [xin]
        for lyr in layers:
            oc = lyr["w"].shape[0]
            args += [lyr["w"], lyr["b"].reshape(oc, 1),
                     lyr["g"].reshape(oc, 1), lyr["beta"].reshape(oc, 1)]
        c_out = layers[-1]["w"].shape[0]
        pooled = pl.pallas_call(
            functools.partial(_mlp_body, K),
            out_shape=jax.ShapeDtypeStruct((c_out, _B * _S), f32),
        )(*args)
        pooled_rows.append(
            pooled.reshape(c_out, _B, _S).transpose(0, 2, 1)
            .reshape(c_out * _S, _B))
    x1 = jnp.concatenate(pooled_rows, axis=0)

    out = pl.pallas_call(
        _head_body,
        out_shape=jax.ShapeDtypeStruct((256, _B), f32),
    )(x1,
      params["fc1_w"], params["fc1_b"].reshape(64, 1),
      params["bn1_g"].reshape(64, 1), params["bn1_b"].reshape(64, 1),
      params["fc2_w"], params["fc2_b"].reshape(256, 1),
      params["bn2_g"].reshape(256, 1), params["bn2_b"].reshape(256, 1))
    return out.T


# speedup vs baseline: 3.1984x; 2.0288x over previous
import functools

import jax
import jax.numpy as jnp
from jax.experimental import pallas as pl

_B = 64
_N = 2048
_S = 16
_RADII = (0.1, 0.2, 0.4)
_NS = (16, 32, 64)


_CHUNK = 16


def _group_body(xyz_ref, o1_ref, o2_ref, o3_ref):
    x = xyz_ref[:, 0, :]
    y = xyz_ref[:, 1, :]
    z = xyz_ref[:, 2, :]
    n0 = xyz_ref[:, 3, :]
    n1 = xyz_ref[:, 4, :]
    n2 = xyz_ref[:, 5, :]
    lane = jax.lax.broadcasted_iota(jnp.int32, (1, _N), 1)
    scol3 = jax.lax.broadcasted_iota(jnp.int32, (1, _S, 1), 1)
    srow3 = jax.lax.broadcasted_iota(jnp.int32, (1, 1, _S), 2)

    def body(i, c):
        dist, far, nc, nr = c
        sel = lane == far
        cx = jnp.sum(jnp.where(sel, x, 0.0), axis=1, keepdims=True)
        cy = jnp.sum(jnp.where(sel, y, 0.0), axis=1, keepdims=True)
        cz = jnp.sum(jnp.where(sel, z, 0.0), axis=1, keepdims=True)
        hitc = scol3 == i
        hitr = srow3 == i
        nc = [jnp.where(hitc, cv.reshape(_CHUNK, 1, 1), old)
              for cv, old in zip((cx, cy, cz), nc)]
        nr = [jnp.where(hitr, cv.reshape(_CHUNK, 1, 1), old)
              for cv, old in zip((cx, cy, cz), nr)]
        dx = x - cx
        dy = y - cy
        dz = z - cz
        d = dx * dx + dy * dy
        d = d + dz * dz
        dist = jnp.minimum(dist, d)
        mx = jnp.max(dist, axis=1, keepdims=True)
        far2 = jnp.min(jnp.where(dist == mx, lane, _N),
                       axis=1, keepdims=True).astype(jnp.int32)
        return dist, far2, nc, nr

    zc = jnp.zeros((_CHUNK, _S, 1), jnp.float32)
    zr = jnp.zeros((_CHUNK, 1, _S), jnp.float32)
    init = (jnp.full((_CHUNK, _N), 1e10, jnp.float32),
            jnp.zeros((_CHUNK, 1), jnp.int32),
            [zc, zc, zc], [zr, zr, zr])
    _, _, (nx3, ny3, nz3), (nxr, nyr, nzr) = jax.lax.fori_loop(
        0, _S, body, init)

    dx3 = x.reshape(_CHUNK, 1, _N) - nx3
    dy3 = y.reshape(_CHUNK, 1, _N) - ny3
    dz3 = z.reshape(_CHUNK, 1, _N) - nz3
    sq3 = dx3 * dx3 + dy3 * dy3
    sq3 = sq3 + dz3 * dz3
    rows = _CHUNK * _S
    sq = sq3.reshape(rows, _N)

    bf = jnp.bfloat16
    xs = (n0, n1, n2, x, y, z)
    his = [v.astype(bf).astype(jnp.float32) for v in xs]
    los = [v - h for v, h in zip(xs, his)]
    nms = (nxr, nyr, nzr)
    nm_hi = [v.astype(bf).astype(jnp.float32) for v in nms]
    nm_lo = [v - h for v, h in zip(nms, nm_hi)]

    for o_ref, radius, K in ((o1_ref, _RADII[0], _NS[0]),
                             (o2_ref, _RADII[1], _NS[1]),
                             (o3_ref, _RADII[2], _NS[2])):
        mask = sq <= jnp.float32(radius ** 2)
        r = mask.astype(jnp.float32)
        sh = 1
        while sh < _N:
            r = r + jnp.concatenate(
                [jnp.zeros((rows, sh), jnp.float32), r[:, : _N - sh]], axis=1)
            sh *= 2
        rm = r - (0.5 - 0.5 * mask.astype(jnp.float32))
        count = r[:, _N - 1:_N]
        kv = jax.lax.broadcasted_iota(
            jnp.int32, (1, K), 1).astype(jnp.float32) + 1.0
        expand = ((jax.lax.broadcasted_iota(jnp.int32, (_S, _S * K), 1) // K)
                  == jax.lax.broadcasted_iota(jnp.int32, (_S, _S * K), 0)
                  ).astype(jnp.float32)

        for si in range(_CHUNK):
            rm_s = rm[si * _S:(si + 1) * _S, :]
            cnt_s = count[si * _S:(si + 1) * _S, :]
            keff = jnp.where(kv <= cnt_s, kv, 1.0)
            selm = (rm_s.reshape(_S, 1, _N) == keff.reshape(_S, K, 1)
                    ).astype(jnp.float32).reshape(_S * K, _N)
            p6hi = jnp.concatenate(
                [h[si:si + 1, :] for h in his], axis=0)
            p6lo = jnp.concatenate(
                [l[si:si + 1, :] for l in los], axis=0)
            dn = (((1,), (1,)), ((), ()))
            feats = (jax.lax.dot_general(
                         p6hi, selm, dn, preferred_element_type=jnp.float32)
                     + jax.lax.dot_general(
                         p6lo, selm, dn, preferred_element_type=jnp.float32))
            nmh = jnp.concatenate(
                [h[si] for h in nm_hi], axis=0)
            nml = jnp.concatenate(
                [l[si] for l in nm_lo], axis=0)
            dm = (((1,), (0,)), ((), ()))
            centers = (jax.lax.dot_general(
                           nmh, expand, dm, preferred_element_type=jnp.float32)
                       + jax.lax.dot_general(
                           nml, expand, dm,
                           preferred_element_type=jnp.float32))
            o_ref[si] = jnp.concatenate(
                [feats[0:3], feats[3:6] - centers], axis=0)


def _mlp_body(K, x_ref, *refs):
    out_ref = refs[-1]
    h = x_ref[...]
    m_cols = h.shape[1]
    for li in range(3):
        w = refs[li * 4][...]
        b = refs[li * 4 + 1][...]
        g = refs[li * 4 + 2][...]
        be = refs[li * 4 + 3][...]
        yv = jax.lax.dot_general(
            w, h, (((1,), (0,)), ((), ())),
            preferred_element_type=jnp.float32) + b
        mu = jnp.mean(yv, axis=1, keepdims=True)
        d = yv - mu
        v = jnp.mean(d * d, axis=1, keepdims=True)
        h = jnp.maximum(d / jnp.sqrt(v + 1e-5) * g + be, 0.0)
    bs = m_cols // K
    p = h[:, 0:bs]
    for k in range(1, K):
        p = jnp.maximum(p, h[:, k * bs:(k + 1) * bs])
    out_ref[...] = p


def _head_body(x_ref, w1_ref, b1_ref, g1_ref, e1_ref,
               w2_ref, b2_ref, g2_ref, e2_ref, out_ref):
    h = x_ref[...]
    y = jax.lax.dot_general(
        w1_ref[...], h, (((1,), (0,)), ((), ())),
        preferred_element_type=jnp.float32) + b1_ref[...]
    mu = jnp.mean(y, axis=1, keepdims=True)
    d = y - mu
    v = jnp.mean(d * d, axis=1, keepdims=True)
    h = jnp.maximum(d / jnp.sqrt(v + 1e-5) * g1_ref[...] + e1_ref[...], 0.0)
    y = jax.lax.dot_general(
        w2_ref[...], h, (((1,), (0,)), ((), ())),
        preferred_element_type=jnp.float32) + b2_ref[...]
    mu = jnp.mean(y, axis=1, keepdims=True)
    d = y - mu
    v = jnp.mean(d * d, axis=1, keepdims=True)
    out_ref[...] = jnp.maximum(
        d / jnp.sqrt(v + 1e-5) * g2_ref[...] + e2_ref[...], 0.0)


def kernel(xyz, params):
    f32 = jnp.float32
    outs1 = pl.pallas_call(
        _group_body,
        grid=(_B // _CHUNK,),
        in_specs=[pl.BlockSpec((_CHUNK, 6, _N), lambda c: (c, 0, 0))],
        out_specs=[pl.BlockSpec((_CHUNK, 6, _S * K), lambda c: (c, 0, 0))
                   for K in _NS],
        out_shape=[jax.ShapeDtypeStruct((_B, 6, _S * K), f32) for K in _NS],
    )(xyz)

    pooled_rows = []
    for i, K in enumerate(_NS):
        xin = (outs1[i].reshape(_B, 6, _S, K)
               .transpose(1, 3, 0, 2).reshape(6, K * _B * _S))
        layers = params["convs"][i]
        args = [xin]
        for lyr in layers:
            oc = lyr["w"].shape[0]
            args += [lyr["w"], lyr["b"].reshape(oc, 1),
                     lyr["g"].reshape(oc, 1), lyr["beta"].reshape(oc, 1)]
        c_out = layers[-1]["w"].shape[0]
        pooled = pl.pallas_call(
            functools.partial(_mlp_body, K),
            out_shape=jax.ShapeDtypeStruct((c_out, _B * _S), f32),
        )(*args)
        pooled_rows.append(
            pooled.reshape(c_out, _B, _S).transpose(0, 2, 1)
            .reshape(c_out * _S, _B))
    x1 = jnp.concatenate(pooled_rows, axis=0)

    out = pl.pallas_call(
        _head_body,
        out_shape=jax.ShapeDtypeStruct((256, _B), f32),
    )(x1,
      params["fc1_w"], params["fc1_b"].reshape(64, 1),
      params["bn1_g"].reshape(64, 1), params["bn1_b"].reshape(64, 1),
      params["fc2_w"], params["fc2_b"].reshape(256, 1),
      params["bn2_g"].reshape(256, 1), params["bn2_b"].reshape(256, 1))
    return out.T


# speedup vs baseline: 3.3147x; 1.0364x over previous
import functools

import jax
import jax.numpy as jnp
from jax.experimental import pallas as pl

_B = 64
_N = 2048
_S = 16
_RADII = (0.1, 0.2, 0.4)
_NS = (16, 32, 64)


_CHUNK = 32


def _group_body(xyz_ref, o1_ref, o2_ref, o3_ref):
    x = xyz_ref[:, 0, :]
    y = xyz_ref[:, 1, :]
    z = xyz_ref[:, 2, :]
    n0 = xyz_ref[:, 3, :]
    n1 = xyz_ref[:, 4, :]
    n2 = xyz_ref[:, 5, :]
    lane = jax.lax.broadcasted_iota(jnp.int32, (1, _N), 1)
    scol3 = jax.lax.broadcasted_iota(jnp.int32, (1, _S, 1), 1)
    srow3 = jax.lax.broadcasted_iota(jnp.int32, (1, 1, _S), 2)

    def body(i, c):
        dist, far, nc, nr = c
        sel = lane == far
        cx = jnp.sum(jnp.where(sel, x, 0.0), axis=1, keepdims=True)
        cy = jnp.sum(jnp.where(sel, y, 0.0), axis=1, keepdims=True)
        cz = jnp.sum(jnp.where(sel, z, 0.0), axis=1, keepdims=True)
        hitc = scol3 == i
        hitr = srow3 == i
        nc = [jnp.where(hitc, cv.reshape(_CHUNK, 1, 1), old)
              for cv, old in zip((cx, cy, cz), nc)]
        nr = [jnp.where(hitr, cv.reshape(_CHUNK, 1, 1), old)
              for cv, old in zip((cx, cy, cz), nr)]
        dx = x - cx
        dy = y - cy
        dz = z - cz
        d = dx * dx + dy * dy
        d = d + dz * dz
        dist = jnp.minimum(dist, d)
        mx = jnp.max(dist, axis=1, keepdims=True)
        far2 = jnp.min(jnp.where(dist == mx, lane, _N),
                       axis=1, keepdims=True).astype(jnp.int32)
        return dist, far2, nc, nr

    zc = jnp.zeros((_CHUNK, _S, 1), jnp.float32)
    zr = jnp.zeros((_CHUNK, 1, _S), jnp.float32)
    init = (jnp.full((_CHUNK, _N), 1e10, jnp.float32),
            jnp.zeros((_CHUNK, 1), jnp.int32),
            [zc, zc, zc], [zr, zr, zr])
    _, _, (nx3, ny3, nz3), (nxr, nyr, nzr) = jax.lax.fori_loop(
        0, _S, body, init)

    dx3 = x.reshape(_CHUNK, 1, _N) - nx3
    dy3 = y.reshape(_CHUNK, 1, _N) - ny3
    dz3 = z.reshape(_CHUNK, 1, _N) - nz3
    sq3 = dx3 * dx3 + dy3 * dy3
    sq3 = sq3 + dz3 * dz3
    rows = _CHUNK * _S
    sq = sq3.reshape(rows, _N)

    bf = jnp.bfloat16
    xs = (n0, n1, n2, x, y, z)
    his = [v.astype(bf).astype(jnp.float32) for v in xs]
    los = [v - h for v, h in zip(xs, his)]
    nms = (nxr, nyr, nzr)
    nm_hi = [v.astype(bf).astype(jnp.float32) for v in nms]
    nm_lo = [v - h for v, h in zip(nms, nm_hi)]

    for o_ref, radius, K in ((o1_ref, _RADII[0], _NS[0]),
                             (o2_ref, _RADII[1], _NS[1]),
                             (o3_ref, _RADII[2], _NS[2])):
        mask = sq <= jnp.float32(radius ** 2)
        r = mask.astype(jnp.float32)
        sh = 1
        while sh < _N:
            r = r + jnp.concatenate(
                [jnp.zeros((rows, sh), jnp.float32), r[:, : _N - sh]], axis=1)
            sh *= 2
        rm = r - (0.5 - 0.5 * mask.astype(jnp.float32))
        count = r[:, _N - 1:_N]
        kv = jax.lax.broadcasted_iota(
            jnp.int32, (1, K), 1).astype(jnp.float32) + 1.0
        expand = ((jax.lax.broadcasted_iota(jnp.int32, (_S, _S * K), 1) // K)
                  == jax.lax.broadcasted_iota(jnp.int32, (_S, _S * K), 0)
                  ).astype(jnp.float32)

        for si in range(_CHUNK):
            rm_s = rm[si * _S:(si + 1) * _S, :]
            cnt_s = count[si * _S:(si + 1) * _S, :]
            keff = jnp.where(kv <= cnt_s, kv, 1.0)
            selm = (rm_s.reshape(_S, 1, _N) == keff.reshape(_S, K, 1)
                    ).astype(jnp.float32).reshape(_S * K, _N)
            p6hi = jnp.concatenate(
                [h[si:si + 1, :] for h in his], axis=0)
            p6lo = jnp.concatenate(
                [l[si:si + 1, :] for l in los], axis=0)
            dn = (((1,), (1,)), ((), ()))
            feats = (jax.lax.dot_general(
                         p6hi, selm, dn, preferred_element_type=jnp.float32)
                     + jax.lax.dot_general(
                         p6lo, selm, dn, preferred_element_type=jnp.float32))
            nmh = jnp.concatenate(
                [h[si] for h in nm_hi], axis=0)
            nml = jnp.concatenate(
                [l[si] for l in nm_lo], axis=0)
            dm = (((1,), (0,)), ((), ()))
            centers = (jax.lax.dot_general(
                           nmh, expand, dm, preferred_element_type=jnp.float32)
                       + jax.lax.dot_general(
                           nml, expand, dm,
                           preferred_element_type=jnp.float32))
            o_ref[si] = jnp.concatenate(
                [feats[0:3], feats[3:6] - centers], axis=0)


def _mlp_body(K, x_ref, *refs):
    out_ref = refs[-1]
    h = x_ref[...]
    m_cols = h.shape[1]
    for li in range(3):
        w = refs[li * 4][...]
        b = refs[li * 4 + 1][...]
        g = refs[li * 4 + 2][...]
        be = refs[li * 4 + 3][...]
        yv = jax.lax.dot_general(
            w, h, (((1,), (0,)), ((), ())),
            preferred_element_type=jnp.float32) + b
        mu = jnp.mean(yv, axis=1, keepdims=True)
        d = yv - mu
        v = jnp.mean(d * d, axis=1, keepdims=True)
        h = jnp.maximum(d / jnp.sqrt(v + 1e-5) * g + be, 0.0)
    bs = m_cols // K
    p = h[:, 0:bs]
    for k in range(1, K):
        p = jnp.maximum(p, h[:, k * bs:(k + 1) * bs])
    out_ref[...] = p


def _head_body(x_ref, w1_ref, b1_ref, g1_ref, e1_ref,
               w2_ref, b2_ref, g2_ref, e2_ref, out_ref):
    h = x_ref[...]
    y = jax.lax.dot_general(
        w1_ref[...], h, (((1,), (0,)), ((), ())),
        preferred_element_type=jnp.float32) + b1_ref[...]
    mu = jnp.mean(y, axis=1, keepdims=True)
    d = y - mu
    v = jnp.mean(d * d, axis=1, keepdims=True)
    h = jnp.maximum(d / jnp.sqrt(v + 1e-5) * g1_ref[...] + e1_ref[...], 0.0)
    y = jax.lax.dot_general(
        w2_ref[...], h, (((1,), (0,)), ((), ())),
        preferred_element_type=jnp.float32) + b2_ref[...]
    mu = jnp.mean(y, axis=1, keepdims=True)
    d = y - mu
    v = jnp.mean(d * d, axis=1, keepdims=True)
    out_ref[...] = jnp.maximum(
        d / jnp.sqrt(v + 1e-5) * g2_ref[...] + e2_ref[...], 0.0)


def kernel(xyz, params):
    f32 = jnp.float32
    outs1 = pl.pallas_call(
        _group_body,
        grid=(_B // _CHUNK,),
        in_specs=[pl.BlockSpec((_CHUNK, 6, _N), lambda c: (c, 0, 0))],
        out_specs=[pl.BlockSpec((_CHUNK, 6, _S * K), lambda c: (c, 0, 0))
                   for K in _NS],
        out_shape=[jax.ShapeDtypeStruct((_B, 6, _S * K), f32) for K in _NS],
    )(xyz)

    pooled_rows = []
    for i, K in enumerate(_NS):
        xin = (outs1[i].reshape(_B, 6, _S, K)
               .transpose(1, 3, 0, 2).reshape(6, K * _B * _S))
        layers = params["convs"][i]
        args = [xin]
        for lyr in layers:
            oc = lyr["w"].shape[0]
            args += [lyr["w"], lyr["b"].reshape(oc, 1),
                     lyr["g"].reshape(oc, 1), lyr["beta"].reshape(oc, 1)]
        c_out = layers[-1]["w"].shape[0]
        pooled = pl.pallas_call(
            functools.partial(_mlp_body, K),
            out_shape=jax.ShapeDtypeStruct((c_out, _B * _S), f32),
        )(*args)
        pooled_rows.append(
            pooled.reshape(c_out, _B, _S).transpose(0, 2, 1)
            .reshape(c_out * _S, _B))
    x1 = jnp.concatenate(pooled_rows, axis=0)

    out = pl.pallas_call(
        _head_body,
        out_shape=jax.ShapeDtypeStruct((256, _B), f32),
    )(x1,
      params["fc1_w"], params["fc1_b"].reshape(64, 1),
      params["bn1_g"].reshape(64, 1), params["bn1_b"].reshape(64, 1),
      params["fc2_w"], params["fc2_b"].reshape(256, 1),
      params["bn2_g"].reshape(256, 1), params["bn2_b"].reshape(256, 1))
    return out.T


# speedup vs baseline: 3.6910x; 1.1135x over previous
import functools

import jax
import jax.numpy as jnp
from jax.experimental import pallas as pl

_B = 64
_N = 2048
_S = 16
_RADII = (0.1, 0.2, 0.4)
_NS = (16, 32, 64)


_CHUNK = 64


def _group_body(xyz_ref, o1_ref, o2_ref, o3_ref):
    x = xyz_ref[:, 0, :]
    y = xyz_ref[:, 1, :]
    z = xyz_ref[:, 2, :]
    n0 = xyz_ref[:, 3, :]
    n1 = xyz_ref[:, 4, :]
    n2 = xyz_ref[:, 5, :]
    lane = jax.lax.broadcasted_iota(jnp.int32, (1, _N), 1)
    scol3 = jax.lax.broadcasted_iota(jnp.int32, (1, _S, 1), 1)
    srow3 = jax.lax.broadcasted_iota(jnp.int32, (1, 1, _S), 2)

    def body(i, c):
        dist, far, nc, nr = c
        sel = lane == far
        cx = jnp.sum(jnp.where(sel, x, 0.0), axis=1, keepdims=True)
        cy = jnp.sum(jnp.where(sel, y, 0.0), axis=1, keepdims=True)
        cz = jnp.sum(jnp.where(sel, z, 0.0), axis=1, keepdims=True)
        hitc = scol3 == i
        hitr = srow3 == i
        nc = [jnp.where(hitc, cv.reshape(_CHUNK, 1, 1), old)
              for cv, old in zip((cx, cy, cz), nc)]
        nr = [jnp.where(hitr, cv.reshape(_CHUNK, 1, 1), old)
              for cv, old in zip((cx, cy, cz), nr)]
        dx = x - cx
        dy = y - cy
        dz = z - cz
        d = dx * dx + dy * dy
        d = d + dz * dz
        dist = jnp.minimum(dist, d)
        mx = jnp.max(dist, axis=1, keepdims=True)
        far2 = jnp.min(jnp.where(dist == mx, lane, _N),
                       axis=1, keepdims=True).astype(jnp.int32)
        return dist, far2, nc, nr

    zc = jnp.zeros((_CHUNK, _S, 1), jnp.float32)
    zr = jnp.zeros((_CHUNK, 1, _S), jnp.float32)
    init = (jnp.full((_CHUNK, _N), 1e10, jnp.float32),
            jnp.zeros((_CHUNK, 1), jnp.int32),
            [zc, zc, zc], [zr, zr, zr])
    _, _, (nx3, ny3, nz3), (nxr, nyr, nzr) = jax.lax.fori_loop(
        0, _S, body, init)

    dx3 = x.reshape(_CHUNK, 1, _N) - nx3
    dy3 = y.reshape(_CHUNK, 1, _N) - ny3
    dz3 = z.reshape(_CHUNK, 1, _N) - nz3
    sq3 = dx3 * dx3 + dy3 * dy3
    sq3 = sq3 + dz3 * dz3
    rows = _CHUNK * _S
    sq = sq3.reshape(rows, _N)

    bf = jnp.bfloat16
    xs = (n0, n1, n2, x, y, z)
    his = [v.astype(bf).astype(jnp.float32) for v in xs]
    los = [v - h for v, h in zip(xs, his)]
    nms = (nxr, nyr, nzr)
    nm_hi = [v.astype(bf).astype(jnp.float32) for v in nms]
    nm_lo = [v - h for v, h in zip(nms, nm_hi)]

    for o_ref, radius, K in ((o1_ref, _RADII[0], _NS[0]),
                             (o2_ref, _RADII[1], _NS[1]),
                             (o3_ref, _RADII[2], _NS[2])):
        mask = sq <= jnp.float32(radius ** 2)
        r = mask.astype(jnp.float32)
        sh = 1
        while sh < _N:
            r = r + jnp.concatenate(
                [jnp.zeros((rows, sh), jnp.float32), r[:, : _N - sh]], axis=1)
            sh *= 2
        rm = r - (0.5 - 0.5 * mask.astype(jnp.float32))
        count = r[:, _N - 1:_N]
        kv = jax.lax.broadcasted_iota(
            jnp.int32, (1, K), 1).astype(jnp.float32) + 1.0
        expand = ((jax.lax.broadcasted_iota(jnp.int32, (_S, _S * K), 1) // K)
                  == jax.lax.broadcasted_iota(jnp.int32, (_S, _S * K), 0)
                  ).astype(jnp.float32)

        for si in range(_CHUNK):
            rm_s = rm[si * _S:(si + 1) * _S, :]
            cnt_s = count[si * _S:(si + 1) * _S, :]
            keff = jnp.where(kv <= cnt_s, kv, 1.0)
            selm = (rm_s.reshape(_S, 1, _N) == keff.reshape(_S, K, 1)
                    ).astype(jnp.float32).reshape(_S * K, _N)
            p6hi = jnp.concatenate(
                [h[si:si + 1, :] for h in his], axis=0)
            p6lo = jnp.concatenate(
                [l[si:si + 1, :] for l in los], axis=0)
            dn = (((1,), (1,)), ((), ()))
            feats = (jax.lax.dot_general(
                         p6hi, selm, dn, preferred_element_type=jnp.float32)
                     + jax.lax.dot_general(
                         p6lo, selm, dn, preferred_element_type=jnp.float32))
            nmh = jnp.concatenate(
                [h[si] for h in nm_hi], axis=0)
            nml = jnp.concatenate(
                [l[si] for l in nm_lo], axis=0)
            dm = (((1,), (0,)), ((), ()))
            centers = (jax.lax.dot_general(
                           nmh, expand, dm, preferred_element_type=jnp.float32)
                       + jax.lax.dot_general(
                           nml, expand, dm,
                           preferred_element_type=jnp.float32))
            o_ref[si] = jnp.concatenate(
                [feats[0:3], feats[3:6] - centers], axis=0)


def _mlp_body(K, x_ref, *refs):
    out_ref = refs[-1]
    h = x_ref[...]
    m_cols = h.shape[1]
    for li in range(3):
        w = refs[li * 4][...]
        b = refs[li * 4 + 1][...]
        g = refs[li * 4 + 2][...]
        be = refs[li * 4 + 3][...]
        yv = jax.lax.dot_general(
            w, h, (((1,), (0,)), ((), ())),
            preferred_element_type=jnp.float32) + b
        mu = jnp.mean(yv, axis=1, keepdims=True)
        d = yv - mu
        v = jnp.mean(d * d, axis=1, keepdims=True)
        h = jnp.maximum(d / jnp.sqrt(v + 1e-5) * g + be, 0.0)
    bs = m_cols // K
    p = h[:, 0:bs]
    for k in range(1, K):
        p = jnp.maximum(p, h[:, k * bs:(k + 1) * bs])
    out_ref[...] = p


def _head_body(x_ref, w1_ref, b1_ref, g1_ref, e1_ref,
               w2_ref, b2_ref, g2_ref, e2_ref, out_ref):
    h = x_ref[...]
    y = jax.lax.dot_general(
        w1_ref[...], h, (((1,), (0,)), ((), ())),
        preferred_element_type=jnp.float32) + b1_ref[...]
    mu = jnp.mean(y, axis=1, keepdims=True)
    d = y - mu
    v = jnp.mean(d * d, axis=1, keepdims=True)
    h = jnp.maximum(d / jnp.sqrt(v + 1e-5) * g1_ref[...] + e1_ref[...], 0.0)
    y = jax.lax.dot_general(
        w2_ref[...], h, (((1,), (0,)), ((), ())),
        preferred_element_type=jnp.float32) + b2_ref[...]
    mu = jnp.mean(y, axis=1, keepdims=True)
    d = y - mu
    v = jnp.mean(d * d, axis=1, keepdims=True)
    out_ref[...] = jnp.maximum(
        d / jnp.sqrt(v + 1e-5) * g2_ref[...] + e2_ref[...], 0.0)


def kernel(xyz, params):
    f32 = jnp.float32
    outs1 = pl.pallas_call(
        _group_body,
        grid=(_B // _CHUNK,),
        in_specs=[pl.BlockSpec((_CHUNK, 6, _N), lambda c: (c, 0, 0))],
        out_specs=[pl.BlockSpec((_CHUNK, 6, _S * K), lambda c: (c, 0, 0))
                   for K in _NS],
        out_shape=[jax.ShapeDtypeStruct((_B, 6, _S * K), f32) for K in _NS],
    )(xyz)

    pooled_rows = []
    for i, K in enumerate(_NS):
        xin = (outs1[i].reshape(_B, 6, _S, K)
               .transpose(1, 3, 0, 2).reshape(6, K * _B * _S))
        layers = params["convs"][i]
        args = [xin]
        for lyr in layers:
            oc = lyr["w"].shape[0]
            args += [lyr["w"], lyr["b"].reshape(oc, 1),
                     lyr["g"].reshape(oc, 1), lyr["beta"].reshape(oc, 1)]
        c_out = layers[-1]["w"].shape[0]
        pooled = pl.pallas_call(
            functools.partial(_mlp_body, K),
            out_shape=jax.ShapeDtypeStruct((c_out, _B * _S), f32),
        )(*args)
        pooled_rows.append(
            pooled.reshape(c_out, _B, _S).transpose(0, 2, 1)
            .reshape(c_out * _S, _B))
    x1 = jnp.concatenate(pooled_rows, axis=0)

    out = pl.pallas_call(
        _head_body,
        out_shape=jax.ShapeDtypeStruct((256, _B), f32),
    )(x1,
      params["fc1_w"], params["fc1_b"].reshape(64, 1),
      params["bn1_g"].reshape(64, 1), params["bn1_b"].reshape(64, 1),
      params["fc2_w"], params["fc2_b"].reshape(256, 1),
      params["bn2_g"].reshape(256, 1), params["bn2_b"].reshape(256, 1))
    return out.T
